# baseline (speedup 1.0000x reference)
import functools

import jax
import jax.numpy as jnp
from jax import lax
from jax.experimental import pallas as pl
from jax.experimental.pallas import tpu as pltpu
from jax.experimental.pallas import tpu_sc as plsc

N = 10000
E = 320000
F = 128
G = 128

NPAD = 10240
TILES = 32
CK = 128
CPT = 80
HALF = CPT // 2
EPAD = TILES * CPT * CK
RPT = NPAD // 16

B = 2000
NB = N // B

_mesh = plsc.VectorSubcoreMesh(core_axis_name="c", subcore_axis_name="s")


@functools.partial(
    pl.kernel,
    out_type=jax.ShapeDtypeStruct((2, NPAD, F), jnp.float32),
    mesh=_mesh,
    scratch_types=[
        pltpu.VMEM((CPT, CK), jnp.int32),
        pltpu.VMEM((CK, F), jnp.float32),
        pltpu.MemorySpace.VMEM_SHARED((NPAD, F), jnp.float32),
    ],
)
def _deg_kernel(colp_hbm, vones_hbm, zeros_hbm, out_hbm, colv, vones, dacc):
    c = lax.axis_index("c")
    s = lax.axis_index("s")
    wid = c * 16 + s
    pltpu.sync_copy(zeros_hbm, dacc.at[pl.ds(s * RPT, RPT)])
    pltpu.sync_copy(colp_hbm.at[wid], colv)
    pltpu.sync_copy(vones_hbm, vones)
    plsc.subcore_barrier()

    def body(j, carry):
        pltpu.sync_copy(vones, dacc.at[colv.at[j]], add=True)
        return carry

    lax.fori_loop(0, CPT, body, 0)
    plsc.subcore_barrier()
    pltpu.sync_copy(dacc.at[pl.ds(s * RPT, RPT)],
                    out_hbm.at[c].at[pl.ds(s * RPT, RPT)])


QTR = 8
CPT0 = 80
CPT1 = 80
NCHUNKS = 16 * (CPT0 + CPT1)


@functools.partial(
    pl.kernel,
    out_type=jax.ShapeDtypeStruct((2, NPAD, F), jnp.float32),
    mesh=_mesh,
    scratch_types=[
        pltpu.VMEM((QTR, CK), jnp.int32),
        pltpu.VMEM((QTR, CK), jnp.int32),
        pltpu.VMEM((CK, F), jnp.float32),
        pltpu.VMEM((CK, F), jnp.float32),
        pltpu.MemorySpace.VMEM_SHARED((NPAD, F), jnp.float32),
        pltpu.SemaphoreType.DMA,
        pltpu.SemaphoreType.DMA,
        pltpu.SemaphoreType.DMA,
        pltpu.SemaphoreType.DMA,
    ],
)
def _agg_kernel(g_hbm, rowp_hbm, colp_hbm, zeros_hbm, out_hbm,
                rowv, colv, bufa, bufb, acc, gsa, gsb, ssa, ssb):
    c = lax.axis_index("c")
    s = lax.axis_index("s")
    base = jnp.where(c == 0, s * CPT0, 16 * CPT0 + s * CPT1)
    nph = jnp.where(c == 0, CPT0 // QTR, CPT1 // QTR)
    pltpu.sync_copy(zeros_hbm, acc.at[pl.ds(s * RPT, RPT)])
    plsc.subcore_barrier()

    def wait_g(sem):
        pltpu.make_async_copy(g_hbm.at[pl.ds(0, CK)], bufa, sem).wait()

    def wait_s(sem):
        pltpu.make_async_copy(bufa, acc.at[pl.ds(0, CK)], sem).wait()

    for q in range(max(CPT0, CPT1) // QTR):
        @pl.when(q < nph)
        def _():
            off = base + q * QTR
            pltpu.sync_copy(rowp_hbm.at[pl.ds(off, QTR)], rowv)
            pltpu.sync_copy(colp_hbm.at[pl.ds(off, QTR)], colv)

            pltpu.async_copy(g_hbm.at[rowv.at[0]], bufa, gsa)
            pltpu.async_copy(g_hbm.at[rowv.at[1]], bufb, gsb)

            def body(i, carry):
                ja = 2 * i
                jb = 2 * i + 1
                wait_g(gsa)
                pltpu.async_copy(bufa, acc.at[colv.at[ja]], ssa, add=True)
                wait_g(gsb)
                pltpu.async_copy(bufb, acc.at[colv.at[jb]], ssb, add=True)

                @pl.when(i < QTR // 2 - 1)
                def _():
                    wait_s(ssa)
                    pltpu.async_copy(g_hbm.at[rowv.at[ja + 2]], bufa, gsa)
                    wait_s(ssb)
                    pltpu.async_copy(g_hbm.at[rowv.at[jb + 2]], bufb, gsb)

                return carry

            lax.fori_loop(0, QTR // 2, body, 0)
            wait_s(ssa)
            wait_s(ssb)

    plsc.subcore_barrier()
    pltpu.sync_copy(acc.at[pl.ds(s * RPT, RPT)],
                    out_hbm.at[c].at[pl.ds(s * RPT, RPT)])


def _dis_of(dp_ref):
    deg = dp_ref[0, :, 0:1] + dp_ref[1, :, 0:1] + 1.0
    return lax.rsqrt(deg)


def _tc1_body(x_ref, w1_ref, w2b_ref, root_ref, dp_ref, g1_ref, dis_ref,
              qg_ref, qacc):
    i = pl.program_id(0)
    dis = _dis_of(dp_ref)
    dis_ref[...] = dis
    xb = x_ref[...]
    h1 = jnp.dot(xb, w1_ref[...], preferred_element_type=jnp.float32)
    g1_ref[...] = h1 * dis
    ids = i * B + lax.broadcasted_iota(jnp.int32, (B, 1), 0)
    ohr = (ids == root_ref[...]).astype(jnp.float32)
    relu_x = jnp.maximum(xb, 0.0)
    contrib = lax.dot_general(ohr, relu_x, (((0,), (0,)), ((), ())),
                              preferred_element_type=jnp.float32)

    @pl.when(i == 0)
    def _():
        qacc[...] = contrib

    @pl.when(i > 0)
    def _():
        qacc[...] = qacc[...] + contrib

    qg_ref[...] = jnp.dot(qacc[...], w2b_ref[...],
                          preferred_element_type=jnp.float32)


def _tc2_body(p_ref, g1_ref, dis_in, b1_ref, qg_ref, batch_ref, w2a_ref,
              c1_ref, g2_ref):
    dis = dis_in[...]
    ssum = p_ref[0] + p_ref[1] + g1_ref[...]
    c1 = dis * ssum + b1_ref[...]
    c1_ref[...] = c1
    z = jnp.maximum(c1, 0.0)
    ohb = (batch_ref[...] == lax.broadcasted_iota(jnp.int32, (1, G), 1)
           ).astype(jnp.float32)
    h2 = (jnp.dot(z, w2a_ref[...], preferred_element_type=jnp.float32)
          + jnp.dot(ohb, qg_ref[...], preferred_element_type=jnp.float32))
    g2_ref[...] = h2 * dis


def _tc3_body(q_ref, g2_ref, dis_in, b2_ref, batch_ref, root_ref, c1_ref,
              out_ref, sums, cnt, r2acc):
    i = pl.program_id(0)
    dis = dis_in[...]
    c2 = dis * (q_ref[0] + q_ref[1] + g2_ref[...]) + b2_ref[...]
    r = jnp.maximum(c2, 0.0)
    ohb = (batch_ref[...] == lax.broadcasted_iota(jnp.int32, (1, G), 1)
           ).astype(jnp.float32)
    s_contrib = lax.dot_general(ohb, r, (((0,), (0,)), ((), ())),
                                preferred_element_type=jnp.float32)
    ones = jnp.ones((B, 1), jnp.float32)
    c_contrib = lax.dot_general(ohb, ones, (((0,), (0,)), ((), ())),
                                preferred_element_type=jnp.float32)
    ids = i * B + lax.broadcasted_iota(jnp.int32, (B, 1), 0)
    ohr = (ids == root_ref[...]).astype(jnp.float32)
    r2_contrib = lax.dot_general(ohr, c1_ref[...], (((0,), (0,)), ((), ())),
                                 preferred_element_type=jnp.float32)

    @pl.when(i == 0)
    def _():
        sums[...] = s_contrib
        cnt[...] = c_contrib
        r2acc[...] = r2_contrib

    @pl.when(i > 0)
    def _():
        sums[...] = sums[...] + s_contrib
        cnt[...] = cnt[...] + c_contrib
        r2acc[...] = r2acc[...] + r2_contrib

    counts = cnt[...]
    o1 = sums[...] / jnp.maximum(counts, 1.0)
    o2 = jnp.where(counts > 0.0, r2acc[...], 0.0)
    out_ref[...] = jnp.concatenate([o1, o2], axis=1)


def _tc1(x, W1, W2b, root2d, dp):
    return pl.pallas_call(
        _tc1_body,
        grid=(NB,),
        in_specs=[
            pl.BlockSpec((B, F), lambda i: (i, 0)),
            pl.BlockSpec((F, F), lambda i: (0, 0)),
            pl.BlockSpec((F, F), lambda i: (0, 0)),
            pl.BlockSpec((1, G), lambda i: (0, 0)),
            pl.BlockSpec((2, B, F), lambda i: (0, i, 0)),
        ],
        out_specs=[
            pl.BlockSpec((B, F), lambda i: (i, 0)),
            pl.BlockSpec((B, 1), lambda i: (i, 0)),
            pl.BlockSpec((G, F), lambda i: (0, 0)),
        ],
        out_shape=[
            jax.ShapeDtypeStruct((N, F), jnp.float32),
            jax.ShapeDtypeStruct((N, 1), jnp.float32),
            jax.ShapeDtypeStruct((G, F), jnp.float32),
        ],
        scratch_shapes=[pltpu.VMEM((G, F), jnp.float32)],
    )(x, W1, W2b, root2d, dp)


def _tc2(p, g1, disb, b1_2d, qg, batch2d, W2a):
    return pl.pallas_call(
        _tc2_body,
        grid=(NB,),
        in_specs=[
            pl.BlockSpec((2, B, F), lambda i: (0, i, 0)),
            pl.BlockSpec((B, F), lambda i: (i, 0)),
            pl.BlockSpec((B, 1), lambda i: (i, 0)),
            pl.BlockSpec((1, F), lambda i: (0, 0)),
            pl.BlockSpec((G, F), lambda i: (0, 0)),
            pl.BlockSpec((B, 1), lambda i: (i, 0)),
            pl.BlockSpec((F, F), lambda i: (0, 0)),
        ],
        out_specs=[
            pl.BlockSpec((B, F), lambda i: (i, 0)),
            pl.BlockSpec((B, F), lambda i: (i, 0)),
        ],
        out_shape=[
            jax.ShapeDtypeStruct((N, F), jnp.float32),
            jax.ShapeDtypeStruct((N, F), jnp.float32),
        ],
    )(p, g1, disb, b1_2d, qg, batch2d, W2a)


def _tc3(q, g2, disb, b2_2d, batch2d, root2d, c1):
    return pl.pallas_call(
        _tc3_body,
        grid=(NB,),
        in_specs=[
            pl.BlockSpec((2, B, F), lambda i: (0, i, 0)),
            pl.BlockSpec((B, F), lambda i: (i, 0)),
            pl.BlockSpec((B, 1), lambda i: (i, 0)),
            pl.BlockSpec((1, F), lambda i: (0, 0)),
            pl.BlockSpec((B, 1), lambda i: (i, 0)),
            pl.BlockSpec((1, G), lambda i: (0, 0)),
            pl.BlockSpec((B, F), lambda i: (i, 0)),
        ],
        out_specs=pl.BlockSpec((G, 2 * F), lambda i: (0, 0)),
        out_shape=jax.ShapeDtypeStruct((G, 2 * F), jnp.float32),
        scratch_shapes=[
            pltpu.VMEM((G, F), jnp.float32),
            pltpu.VMEM((G, 1), jnp.float32),
            pltpu.VMEM((G, F), jnp.float32),
        ],
    )(q, g2, disb, b2_2d, batch2d, root2d, c1)


@jax.jit
def kernel(x, edge_index, batch, root_index, W1, b1, W2, b2):
    row = edge_index[0].astype(jnp.int32)
    col = edge_index[1].astype(jnp.int32)
    pad = EPAD - E
    lane = jnp.arange(CK, dtype=jnp.int32)
    padrow = jnp.tile(lane * 77, pad // CK)
    padcol = jnp.tile(N + lane, pad // CK)
    rowp = jnp.concatenate([row, padrow])
    colp = jnp.concatenate([col, padcol])
    rowp = rowp.reshape(NCHUNKS, CK)
    colp = colp.reshape(NCHUNKS, CK)
    colp_deg = colp.reshape(TILES, CPT, CK)

    vones = jnp.zeros((CK, F), jnp.float32).at[:, 0].set(1.0)
    zerosF = jnp.zeros((RPT, F), jnp.float32)

    root2d = root_index.astype(jnp.int32).reshape(1, G)
    batch2d = batch.astype(jnp.int32).reshape(N, 1)
    b1_2d = b1.reshape(1, F)
    b2_2d = b2.reshape(1, F)
    W2a = W2[:F]
    W2b = W2[F:]

    dp = _deg_kernel(colp_deg, vones, zerosF)
    g1, disb, qg = _tc1(x, W1, W2b, root2d, dp)
    p1 = _agg_kernel(g1, rowp, colp, zerosF)
    c1, g2 = _tc2(p1, g1, disb, b1_2d, qg, batch2d, W2a)
    p2 = _agg_kernel(g2, rowp, colp, zerosF)
    out = _tc3(p2, g2, disb, b2_2d, batch2d, root2d, c1)
    return out


# speedup vs baseline: 1.1293x; 1.1293x over previous
import functools

import jax
import jax.numpy as jnp
from jax import lax
from jax.experimental import pallas as pl
from jax.experimental.pallas import tpu as pltpu
from jax.experimental.pallas import tpu_sc as plsc

N = 10000
E = 320000
F = 128
G = 128

NPAD = 10240
TILES = 32
CK = 128
CPT = 80
HALF = CPT // 2
EPAD = TILES * CPT * CK
RPT = NPAD // 16

B = 2000
NB = N // B

_mesh = plsc.VectorSubcoreMesh(core_axis_name="c", subcore_axis_name="s")


@functools.partial(
    pl.kernel,
    out_type=jax.ShapeDtypeStruct((2, NPAD, F), jnp.float32),
    mesh=_mesh,
    scratch_types=[
        pltpu.VMEM((CPT, CK), jnp.int32),
        pltpu.VMEM((CK, F), jnp.float32),
        pltpu.MemorySpace.VMEM_SHARED((NPAD, F), jnp.float32),
    ],
)
def _deg_kernel(colp_hbm, vones_hbm, zeros_hbm, out_hbm, colv, vones, dacc):
    c = lax.axis_index("c")
    s = lax.axis_index("s")
    wid = c * 16 + s
    pltpu.sync_copy(zeros_hbm, dacc.at[pl.ds(s * RPT, RPT)])
    pltpu.sync_copy(colp_hbm.at[wid], colv)
    pltpu.sync_copy(vones_hbm, vones)
    plsc.subcore_barrier()

    def body(j, carry):
        pltpu.sync_copy(vones, dacc.at[colv.at[j]], add=True)
        return carry

    lax.fori_loop(0, CPT, body, 0)
    plsc.subcore_barrier()
    pltpu.sync_copy(dacc.at[pl.ds(s * RPT, RPT)],
                    out_hbm.at[c].at[pl.ds(s * RPT, RPT)])


QTR = 8
CPT0 = 80
CPT1 = 80
NCHUNKS = 16 * (CPT0 + CPT1)


@functools.partial(
    pl.kernel,
    out_type=jax.ShapeDtypeStruct((2, NPAD, F), jnp.float32),
    mesh=_mesh,
    scratch_types=[
        pltpu.VMEM((QTR, CK), jnp.int32),
        pltpu.VMEM((QTR, CK), jnp.int32),
        pltpu.VMEM((CK, F), jnp.float32),
        pltpu.VMEM((CK, F), jnp.float32),
        pltpu.MemorySpace.VMEM_SHARED((NPAD, F), jnp.float32),
        pltpu.SemaphoreType.DMA,
        pltpu.SemaphoreType.DMA,
    ],
)
def _agg_kernel(g_hbm, rowp_hbm, colp_hbm, zeros_hbm, out_hbm,
                rowv, colv, bufa, bufb, acc, sema, semb):
    c = lax.axis_index("c")
    s = lax.axis_index("s")
    base = jnp.where(c == 0, s * CPT0, 16 * CPT0 + s * CPT1)
    nph = jnp.where(c == 0, CPT0 // QTR, CPT1 // QTR)
    pltpu.sync_copy(zeros_hbm, acc.at[pl.ds(s * RPT, RPT)])
    plsc.subcore_barrier()

    def wait_g(sem):
        pltpu.make_async_copy(g_hbm.at[pl.ds(0, CK)], bufa, sem).wait()

    for q in range(max(CPT0, CPT1) // QTR):
        @pl.when(q < nph)
        def _():
            off = base + q * QTR
            pltpu.sync_copy(rowp_hbm.at[pl.ds(off, QTR)], rowv)
            pltpu.sync_copy(colp_hbm.at[pl.ds(off, QTR)], colv)

            pltpu.async_copy(g_hbm.at[rowv.at[0]], bufa, sema)

            def body(i, carry):
                ja = 2 * i
                jb = 2 * i + 1
                pltpu.async_copy(g_hbm.at[rowv.at[jb]], bufb, semb)
                wait_g(sema)
                pltpu.sync_copy(bufa, acc.at[colv.at[ja]], add=True)

                @pl.when(i < QTR // 2 - 1)
                def _():
                    pltpu.async_copy(g_hbm.at[rowv.at[ja + 2]], bufa, sema)

                wait_g(semb)
                pltpu.sync_copy(bufb, acc.at[colv.at[jb]], add=True)
                return carry

            lax.fori_loop(0, QTR // 2, body, 0)

    plsc.subcore_barrier()
    pltpu.sync_copy(acc.at[pl.ds(s * RPT, RPT)],
                    out_hbm.at[c].at[pl.ds(s * RPT, RPT)])


def _dis_of(dp_ref):
    deg = dp_ref[0, :, 0:1] + dp_ref[1, :, 0:1] + 1.0
    return lax.rsqrt(deg)


def _tc1_body(x_ref, w1_ref, w2b_ref, root_ref, dp_ref, g1_ref, dis_ref,
              qg_ref, qacc):
    i = pl.program_id(0)
    dis = _dis_of(dp_ref)
    dis_ref[...] = dis
    xb = x_ref[...]
    h1 = jnp.dot(xb, w1_ref[...], preferred_element_type=jnp.float32)
    g1_ref[...] = h1 * dis
    ids = i * B + lax.broadcasted_iota(jnp.int32, (B, 1), 0)
    ohr = (ids == root_ref[...]).astype(jnp.float32)
    relu_x = jnp.maximum(xb, 0.0)
    contrib = lax.dot_general(ohr, relu_x, (((0,), (0,)), ((), ())),
                              preferred_element_type=jnp.float32)

    @pl.when(i == 0)
    def _():
        qacc[...] = contrib

    @pl.when(i > 0)
    def _():
        qacc[...] = qacc[...] + contrib

    qg_ref[...] = jnp.dot(qacc[...], w2b_ref[...],
                          preferred_element_type=jnp.float32)


def _tc2_body(p_ref, g1_ref, dis_in, b1_ref, qg_ref, batch_ref, w2a_ref,
              c1_ref, g2_ref):
    dis = dis_in[...]
    ssum = p_ref[0] + p_ref[1] + g1_ref[...]
    c1 = dis * ssum + b1_ref[...]
    c1_ref[...] = c1
    z = jnp.maximum(c1, 0.0)
    ohb = (batch_ref[...] == lax.broadcasted_iota(jnp.int32, (1, G), 1)
           ).astype(jnp.float32)
    h2 = (jnp.dot(z, w2a_ref[...], preferred_element_type=jnp.float32)
          + jnp.dot(ohb, qg_ref[...], preferred_element_type=jnp.float32))
    g2_ref[...] = h2 * dis


def _tc3_body(q_ref, g2_ref, dis_in, b2_ref, batch_ref, root_ref, c1_ref,
              out_ref, sums, cnt, r2acc):
    i = pl.program_id(0)
    dis = dis_in[...]
    c2 = dis * (q_ref[0] + q_ref[1] + g2_ref[...]) + b2_ref[...]
    r = jnp.maximum(c2, 0.0)
    ohb = (batch_ref[...] == lax.broadcasted_iota(jnp.int32, (1, G), 1)
           ).astype(jnp.float32)
    s_contrib = lax.dot_general(ohb, r, (((0,), (0,)), ((), ())),
                                preferred_element_type=jnp.float32)
    ones = jnp.ones((B, 1), jnp.float32)
    c_contrib = lax.dot_general(ohb, ones, (((0,), (0,)), ((), ())),
                                preferred_element_type=jnp.float32)
    ids = i * B + lax.broadcasted_iota(jnp.int32, (B, 1), 0)
    ohr = (ids == root_ref[...]).astype(jnp.float32)
    r2_contrib = lax.dot_general(ohr, c1_ref[...], (((0,), (0,)), ((), ())),
                                 preferred_element_type=jnp.float32)

    @pl.when(i == 0)
    def _():
        sums[...] = s_contrib
        cnt[...] = c_contrib
        r2acc[...] = r2_contrib

    @pl.when(i > 0)
    def _():
        sums[...] = sums[...] + s_contrib
        cnt[...] = cnt[...] + c_contrib
        r2acc[...] = r2acc[...] + r2_contrib

    counts = cnt[...]
    o1 = sums[...] / jnp.maximum(counts, 1.0)
    o2 = jnp.where(counts > 0.0, r2acc[...], 0.0)
    out_ref[...] = jnp.concatenate([o1, o2], axis=1)


def _tc1(x, W1, W2b, root2d, dp):
    return pl.pallas_call(
        _tc1_body,
        grid=(NB,),
        in_specs=[
            pl.BlockSpec((B, F), lambda i: (i, 0)),
            pl.BlockSpec((F, F), lambda i: (0, 0)),
            pl.BlockSpec((F, F), lambda i: (0, 0)),
            pl.BlockSpec((1, G), lambda i: (0, 0)),
            pl.BlockSpec((2, B, F), lambda i: (0, i, 0)),
        ],
        out_specs=[
            pl.BlockSpec((B, F), lambda i: (i, 0)),
            pl.BlockSpec((B, 1), lambda i: (i, 0)),
            pl.BlockSpec((G, F), lambda i: (0, 0)),
        ],
        out_shape=[
            jax.ShapeDtypeStruct((N, F), jnp.float32),
            jax.ShapeDtypeStruct((N, 1), jnp.float32),
            jax.ShapeDtypeStruct((G, F), jnp.float32),
        ],
        scratch_shapes=[pltpu.VMEM((G, F), jnp.float32)],
    )(x, W1, W2b, root2d, dp)


def _tc2(p, g1, disb, b1_2d, qg, batch2d, W2a):
    return pl.pallas_call(
        _tc2_body,
        grid=(NB,),
        in_specs=[
            pl.BlockSpec((2, B, F), lambda i: (0, i, 0)),
            pl.BlockSpec((B, F), lambda i: (i, 0)),
            pl.BlockSpec((B, 1), lambda i: (i, 0)),
            pl.BlockSpec((1, F), lambda i: (0, 0)),
            pl.BlockSpec((G, F), lambda i: (0, 0)),
            pl.BlockSpec((B, 1), lambda i: (i, 0)),
            pl.BlockSpec((F, F), lambda i: (0, 0)),
        ],
        out_specs=[
            pl.BlockSpec((B, F), lambda i: (i, 0)),
            pl.BlockSpec((B, F), lambda i: (i, 0)),
        ],
        out_shape=[
            jax.ShapeDtypeStruct((N, F), jnp.float32),
            jax.ShapeDtypeStruct((N, F), jnp.float32),
        ],
    )(p, g1, disb, b1_2d, qg, batch2d, W2a)


def _tc3(q, g2, disb, b2_2d, batch2d, root2d, c1):
    return pl.pallas_call(
        _tc3_body,
        grid=(NB,),
        in_specs=[
            pl.BlockSpec((2, B, F), lambda i: (0, i, 0)),
            pl.BlockSpec((B, F), lambda i: (i, 0)),
            pl.BlockSpec((B, 1), lambda i: (i, 0)),
            pl.BlockSpec((1, F), lambda i: (0, 0)),
            pl.BlockSpec((B, 1), lambda i: (i, 0)),
            pl.BlockSpec((1, G), lambda i: (0, 0)),
            pl.BlockSpec((B, F), lambda i: (i, 0)),
        ],
        out_specs=pl.BlockSpec((G, 2 * F), lambda i: (0, 0)),
        out_shape=jax.ShapeDtypeStruct((G, 2 * F), jnp.float32),
        scratch_shapes=[
            pltpu.VMEM((G, F), jnp.float32),
            pltpu.VMEM((G, 1), jnp.float32),
            pltpu.VMEM((G, F), jnp.float32),
        ],
    )(q, g2, disb, b2_2d, batch2d, root2d, c1)


@jax.jit
def kernel(x, edge_index, batch, root_index, W1, b1, W2, b2):
    row = edge_index[0].astype(jnp.int32)
    col = edge_index[1].astype(jnp.int32)
    pad = EPAD - E
    lane = jnp.arange(CK, dtype=jnp.int32)
    padrow = jnp.tile(lane * 77, pad // CK)
    padcol = jnp.tile(N + lane, pad // CK)
    rowp = jnp.concatenate([row, padrow])
    colp = jnp.concatenate([col, padcol])
    rowp = rowp.reshape(NCHUNKS, CK)
    colp = colp.reshape(NCHUNKS, CK)
    colp_deg = colp.reshape(TILES, CPT, CK)

    vones = jnp.zeros((CK, F), jnp.float32).at[:, 0].set(1.0)
    zerosF = jnp.zeros((RPT, F), jnp.float32)

    root2d = root_index.astype(jnp.int32).reshape(1, G)
    batch2d = batch.astype(jnp.int32).reshape(N, 1)
    b1_2d = b1.reshape(1, F)
    b2_2d = b2.reshape(1, F)
    W2a = W2[:F]
    W2b = W2[F:]

    dp = _deg_kernel(colp_deg, vones, zerosF)
    g1, disb, qg = _tc1(x, W1, W2b, root2d, dp)
    p1 = _agg_kernel(g1, rowp, colp, zerosF)
    c1, g2 = _tc2(p1, g1, disb, b1_2d, qg, batch2d, W2a)
    p2 = _agg_kernel(g2, rowp, colp, zerosF)
    out = _tc3(p2, g2, disb, b2_2d, batch2d, root2d, c1)
    return out


# speedup vs baseline: 1.1965x; 1.0595x over previous
import functools

import jax
import jax.numpy as jnp
from jax import lax
from jax.experimental import pallas as pl
from jax.experimental.pallas import tpu as pltpu
from jax.experimental.pallas import tpu_sc as plsc

N = 10000
E = 320000
F = 128
G = 128

NPAD = 10240
TILES = 32
CK = 128
CPT = 80
HALF = CPT // 2
EPAD = TILES * CPT * CK
RPT = NPAD // 16

B = 2000
NB = N // B

_mesh = plsc.VectorSubcoreMesh(core_axis_name="c", subcore_axis_name="s")


@functools.partial(
    pl.kernel,
    out_type=jax.ShapeDtypeStruct((2, NPAD, F), jnp.float32),
    mesh=_mesh,
    scratch_types=[
        pltpu.VMEM((CPT, CK), jnp.int32),
        pltpu.VMEM((CK, F), jnp.float32),
        pltpu.MemorySpace.VMEM_SHARED((NPAD, F), jnp.float32),
    ],
)
def _deg_kernel(colp_hbm, vones_hbm, zeros_hbm, out_hbm, colv, vones, dacc):
    c = lax.axis_index("c")
    s = lax.axis_index("s")
    wid = c * 16 + s
    pltpu.sync_copy(zeros_hbm, dacc.at[pl.ds(s * RPT, RPT)])
    pltpu.sync_copy(colp_hbm.at[wid], colv)
    pltpu.sync_copy(vones_hbm, vones)
    plsc.subcore_barrier()

    def body(j, carry):
        pltpu.sync_copy(vones, dacc.at[colv.at[j]], add=True)
        return carry

    lax.fori_loop(0, CPT, body, 0)
    plsc.subcore_barrier()
    pltpu.sync_copy(dacc.at[pl.ds(s * RPT, RPT)],
                    out_hbm.at[c].at[pl.ds(s * RPT, RPT)])


QTR = 16
CPT0 = 80
CPT1 = 80
NCHUNKS = 16 * (CPT0 + CPT1)


@functools.partial(
    pl.kernel,
    out_type=jax.ShapeDtypeStruct((2, NPAD, F), jnp.float32),
    mesh=_mesh,
    scratch_types=[
        pltpu.VMEM((QTR, CK), jnp.int32),
        pltpu.VMEM((QTR, CK), jnp.int32),
        pltpu.VMEM((CK, F), jnp.float32),
        pltpu.VMEM((CK, F), jnp.float32),
        pltpu.MemorySpace.VMEM_SHARED((NPAD, F), jnp.float32),
        pltpu.SemaphoreType.DMA,
        pltpu.SemaphoreType.DMA,
    ],
)
def _agg_kernel(g_hbm, rowp_hbm, colp_hbm, zeros_hbm, out_hbm,
                rowv, colv, bufa, bufb, acc, sema, semb):
    c = lax.axis_index("c")
    s = lax.axis_index("s")
    base = jnp.where(c == 0, s * CPT0, 16 * CPT0 + s * CPT1)
    nph = jnp.where(c == 0, CPT0 // QTR, CPT1 // QTR)
    pltpu.sync_copy(zeros_hbm, acc.at[pl.ds(s * RPT, RPT)])
    plsc.subcore_barrier()

    def wait_g(sem):
        pltpu.make_async_copy(g_hbm.at[pl.ds(0, CK)], bufa, sem).wait()

    for q in range(max(CPT0, CPT1) // QTR):
        @pl.when(q < nph)
        def _():
            off = base + q * QTR
            pltpu.sync_copy(rowp_hbm.at[pl.ds(off, QTR)], rowv)
            pltpu.sync_copy(colp_hbm.at[pl.ds(off, QTR)], colv)

            pltpu.async_copy(g_hbm.at[rowv.at[0]], bufa, sema)

            def body(i, carry):
                ja = 2 * i
                jb = 2 * i + 1
                pltpu.async_copy(g_hbm.at[rowv.at[jb]], bufb, semb)
                wait_g(sema)
                pltpu.sync_copy(bufa, acc.at[colv.at[ja]], add=True)

                @pl.when(i < QTR // 2 - 1)
                def _():
                    pltpu.async_copy(g_hbm.at[rowv.at[ja + 2]], bufa, sema)

                wait_g(semb)
                pltpu.sync_copy(bufb, acc.at[colv.at[jb]], add=True)
                return carry

            lax.fori_loop(0, QTR // 2, body, 0)

    plsc.subcore_barrier()
    pltpu.sync_copy(acc.at[pl.ds(s * RPT, RPT)],
                    out_hbm.at[c].at[pl.ds(s * RPT, RPT)])


def _dis_of(dp_ref):
    deg = dp_ref[0, :, 0:1] + dp_ref[1, :, 0:1] + 1.0
    return lax.rsqrt(deg)


def _tc1_body(x_ref, w1_ref, w2b_ref, root_ref, dp_ref, g1_ref, dis_ref,
              qg_ref, qacc):
    i = pl.program_id(0)
    dis = _dis_of(dp_ref)
    dis_ref[...] = dis
    xb = x_ref[...]
    h1 = jnp.dot(xb, w1_ref[...], preferred_element_type=jnp.float32)
    g1_ref[...] = h1 * dis
    ids = i * B + lax.broadcasted_iota(jnp.int32, (B, 1), 0)
    ohr = (ids == root_ref[...]).astype(jnp.float32)
    relu_x = jnp.maximum(xb, 0.0)
    contrib = lax.dot_general(ohr, relu_x, (((0,), (0,)), ((), ())),
                              preferred_element_type=jnp.float32)

    @pl.when(i == 0)
    def _():
        qacc[...] = contrib

    @pl.when(i > 0)
    def _():
        qacc[...] = qacc[...] + contrib

    qg_ref[...] = jnp.dot(qacc[...], w2b_ref[...],
                          preferred_element_type=jnp.float32)


def _tc2_body(p_ref, g1_ref, dis_in, b1_ref, qg_ref, batch_ref, w2a_ref,
              c1_ref, g2_ref):
    dis = dis_in[...]
    ssum = p_ref[0] + p_ref[1] + g1_ref[...]
    c1 = dis * ssum + b1_ref[...]
    c1_ref[...] = c1
    z = jnp.maximum(c1, 0.0)
    ohb = (batch_ref[...] == lax.broadcasted_iota(jnp.int32, (1, G), 1)
           ).astype(jnp.float32)
    h2 = (jnp.dot(z, w2a_ref[...], preferred_element_type=jnp.float32)
          + jnp.dot(ohb, qg_ref[...], preferred_element_type=jnp.float32))
    g2_ref[...] = h2 * dis


def _tc3_body(q_ref, g2_ref, dis_in, b2_ref, batch_ref, root_ref, c1_ref,
              out_ref, sums, cnt, r2acc):
    i = pl.program_id(0)
    dis = dis_in[...]
    c2 = dis * (q_ref[0] + q_ref[1] + g2_ref[...]) + b2_ref[...]
    r = jnp.maximum(c2, 0.0)
    ohb = (batch_ref[...] == lax.broadcasted_iota(jnp.int32, (1, G), 1)
           ).astype(jnp.float32)
    s_contrib = lax.dot_general(ohb, r, (((0,), (0,)), ((), ())),
                                preferred_element_type=jnp.float32)
    ones = jnp.ones((B, 1), jnp.float32)
    c_contrib = lax.dot_general(ohb, ones, (((0,), (0,)), ((), ())),
                                preferred_element_type=jnp.float32)
    ids = i * B + lax.broadcasted_iota(jnp.int32, (B, 1), 0)
    ohr = (ids == root_ref[...]).astype(jnp.float32)
    r2_contrib = lax.dot_general(ohr, c1_ref[...], (((0,), (0,)), ((), ())),
                                 preferred_element_type=jnp.float32)

    @pl.when(i == 0)
    def _():
        sums[...] = s_contrib
        cnt[...] = c_contrib
        r2acc[...] = r2_contrib

    @pl.when(i > 0)
    def _():
        sums[...] = sums[...] + s_contrib
        cnt[...] = cnt[...] + c_contrib
        r2acc[...] = r2acc[...] + r2_contrib

    counts = cnt[...]
    o1 = sums[...] / jnp.maximum(counts, 1.0)
    o2 = jnp.where(counts > 0.0, r2acc[...], 0.0)
    out_ref[...] = jnp.concatenate([o1, o2], axis=1)


def _tc1(x, W1, W2b, root2d, dp):
    return pl.pallas_call(
        _tc1_body,
        grid=(NB,),
        in_specs=[
            pl.BlockSpec((B, F), lambda i: (i, 0)),
            pl.BlockSpec((F, F), lambda i: (0, 0)),
            pl.BlockSpec((F, F), lambda i: (0, 0)),
            pl.BlockSpec((1, G), lambda i: (0, 0)),
            pl.BlockSpec((2, B, F), lambda i: (0, i, 0)),
        ],
        out_specs=[
            pl.BlockSpec((B, F), lambda i: (i, 0)),
            pl.BlockSpec((B, 1), lambda i: (i, 0)),
            pl.BlockSpec((G, F), lambda i: (0, 0)),
        ],
        out_shape=[
            jax.ShapeDtypeStruct((N, F), jnp.float32),
            jax.ShapeDtypeStruct((N, 1), jnp.float32),
            jax.ShapeDtypeStruct((G, F), jnp.float32),
        ],
        scratch_shapes=[pltpu.VMEM((G, F), jnp.float32)],
    )(x, W1, W2b, root2d, dp)


def _tc2(p, g1, disb, b1_2d, qg, batch2d, W2a):
    return pl.pallas_call(
        _tc2_body,
        grid=(NB,),
        in_specs=[
            pl.BlockSpec((2, B, F), lambda i: (0, i, 0)),
            pl.BlockSpec((B, F), lambda i: (i, 0)),
            pl.BlockSpec((B, 1), lambda i: (i, 0)),
            pl.BlockSpec((1, F), lambda i: (0, 0)),
            pl.BlockSpec((G, F), lambda i: (0, 0)),
            pl.BlockSpec((B, 1), lambda i: (i, 0)),
            pl.BlockSpec((F, F), lambda i: (0, 0)),
        ],
        out_specs=[
            pl.BlockSpec((B, F), lambda i: (i, 0)),
            pl.BlockSpec((B, F), lambda i: (i, 0)),
        ],
        out_shape=[
            jax.ShapeDtypeStruct((N, F), jnp.float32),
            jax.ShapeDtypeStruct((N, F), jnp.float32),
        ],
    )(p, g1, disb, b1_2d, qg, batch2d, W2a)


def _tc3(q, g2, disb, b2_2d, batch2d, root2d, c1):
    return pl.pallas_call(
        _tc3_body,
        grid=(NB,),
        in_specs=[
            pl.BlockSpec((2, B, F), lambda i: (0, i, 0)),
            pl.BlockSpec((B, F), lambda i: (i, 0)),
            pl.BlockSpec((B, 1), lambda i: (i, 0)),
            pl.BlockSpec((1, F), lambda i: (0, 0)),
            pl.BlockSpec((B, 1), lambda i: (i, 0)),
            pl.BlockSpec((1, G), lambda i: (0, 0)),
            pl.BlockSpec((B, F), lambda i: (i, 0)),
        ],
        out_specs=pl.BlockSpec((G, 2 * F), lambda i: (0, 0)),
        out_shape=jax.ShapeDtypeStruct((G, 2 * F), jnp.float32),
        scratch_shapes=[
            pltpu.VMEM((G, F), jnp.float32),
            pltpu.VMEM((G, 1), jnp.float32),
            pltpu.VMEM((G, F), jnp.float32),
        ],
    )(q, g2, disb, b2_2d, batch2d, root2d, c1)


@jax.jit
def kernel(x, edge_index, batch, root_index, W1, b1, W2, b2):
    row = edge_index[0].astype(jnp.int32)
    col = edge_index[1].astype(jnp.int32)
    pad = EPAD - E
    lane = jnp.arange(CK, dtype=jnp.int32)
    padrow = jnp.tile(lane * 77, pad // CK)
    padcol = jnp.tile(N + lane, pad // CK)
    rowp = jnp.concatenate([row, padrow])
    colp = jnp.concatenate([col, padcol])
    rowp = rowp.reshape(NCHUNKS, CK)
    colp = colp.reshape(NCHUNKS, CK)
    colp_deg = colp.reshape(TILES, CPT, CK)

    vones = jnp.zeros((CK, F), jnp.float32).at[:, 0].set(1.0)
    zerosF = jnp.zeros((RPT, F), jnp.float32)

    root2d = root_index.astype(jnp.int32).reshape(1, G)
    batch2d = batch.astype(jnp.int32).reshape(N, 1)
    b1_2d = b1.reshape(1, F)
    b2_2d = b2.reshape(1, F)
    W2a = W2[:F]
    W2b = W2[F:]

    dp = _deg_kernel(colp_deg, vones, zerosF)
    g1, disb, qg = _tc1(x, W1, W2b, root2d, dp)
    p1 = _agg_kernel(g1, rowp, colp, zerosF)
    c1, g2 = _tc2(p1, g1, disb, b1_2d, qg, batch2d, W2a)
    p2 = _agg_kernel(g2, rowp, colp, zerosF)
    out = _tc3(p2, g2, disb, b2_2d, batch2d, root2d, c1)
    return out


# speedup vs baseline: 1.2611x; 1.0540x over previous
import functools

import jax
import jax.numpy as jnp
from jax import lax
from jax.experimental import pallas as pl
from jax.experimental.pallas import tpu as pltpu
from jax.experimental.pallas import tpu_sc as plsc

N = 10000
E = 320000
F = 128
G = 128

NPAD = 10240
TILES = 32
CK = 128
CPT = 80
HALF = CPT // 2
EPAD = TILES * CPT * CK
RPT = NPAD // 16

B = 2000
NB = N // B

_mesh = plsc.VectorSubcoreMesh(core_axis_name="c", subcore_axis_name="s")


@functools.partial(
    pl.kernel,
    out_type=jax.ShapeDtypeStruct((2, NPAD, F), jnp.float32),
    mesh=_mesh,
    scratch_types=[
        pltpu.VMEM((CPT, CK), jnp.int32),
        pltpu.VMEM((CK, F), jnp.float32),
        pltpu.MemorySpace.VMEM_SHARED((NPAD, F), jnp.float32),
    ],
)
def _deg_kernel(colp_hbm, vones_hbm, zeros_hbm, out_hbm, colv, vones, dacc):
    c = lax.axis_index("c")
    s = lax.axis_index("s")
    wid = c * 16 + s
    pltpu.sync_copy(zeros_hbm, dacc.at[pl.ds(s * RPT, RPT)])
    pltpu.sync_copy(colp_hbm.at[wid], colv)
    pltpu.sync_copy(vones_hbm, vones)
    plsc.subcore_barrier()

    def body(j, carry):
        pltpu.sync_copy(vones, dacc.at[colv.at[j]], add=True)
        return carry

    lax.fori_loop(0, CPT, body, 0)
    plsc.subcore_barrier()
    pltpu.sync_copy(dacc.at[pl.ds(s * RPT, RPT)],
                    out_hbm.at[c].at[pl.ds(s * RPT, RPT)])


QTR = 40
CPT0 = 80
CPT1 = 80
NCHUNKS = 16 * (CPT0 + CPT1)


@functools.partial(
    pl.kernel,
    out_type=jax.ShapeDtypeStruct((2, NPAD, F), jnp.float32),
    mesh=_mesh,
    scratch_types=[
        pltpu.VMEM((QTR, CK), jnp.int32),
        pltpu.VMEM((QTR, CK), jnp.int32),
        pltpu.VMEM((CK, F), jnp.float32),
        pltpu.VMEM((CK, F), jnp.float32),
        pltpu.MemorySpace.VMEM_SHARED((NPAD, F), jnp.float32),
        pltpu.SemaphoreType.DMA,
        pltpu.SemaphoreType.DMA,
    ],
)
def _agg_kernel(g_hbm, rowp_hbm, colp_hbm, zeros_hbm, out_hbm,
                rowv, colv, bufa, bufb, acc, sema, semb):
    c = lax.axis_index("c")
    s = lax.axis_index("s")
    base = jnp.where(c == 0, s * CPT0, 16 * CPT0 + s * CPT1)
    nph = jnp.where(c == 0, CPT0 // QTR, CPT1 // QTR)
    pltpu.sync_copy(zeros_hbm, acc.at[pl.ds(s * RPT, RPT)])
    plsc.subcore_barrier()

    def wait_g(sem):
        pltpu.make_async_copy(g_hbm.at[pl.ds(0, CK)], bufa, sem).wait()

    for q in range(max(CPT0, CPT1) // QTR):
        @pl.when(q < nph)
        def _():
            off = base + q * QTR
            pltpu.sync_copy(rowp_hbm.at[pl.ds(off, QTR)], rowv)
            pltpu.sync_copy(colp_hbm.at[pl.ds(off, QTR)], colv)

            pltpu.async_copy(g_hbm.at[rowv.at[0]], bufa, sema)

            def body(i, carry):
                ja = 2 * i
                jb = 2 * i + 1
                pltpu.async_copy(g_hbm.at[rowv.at[jb]], bufb, semb)
                wait_g(sema)
                pltpu.sync_copy(bufa, acc.at[colv.at[ja]], add=True)

                @pl.when(i < QTR // 2 - 1)
                def _():
                    pltpu.async_copy(g_hbm.at[rowv.at[ja + 2]], bufa, sema)

                wait_g(semb)
                pltpu.sync_copy(bufb, acc.at[colv.at[jb]], add=True)
                return carry

            lax.fori_loop(0, QTR // 2, body, 0)

    plsc.subcore_barrier()
    pltpu.sync_copy(acc.at[pl.ds(s * RPT, RPT)],
                    out_hbm.at[c].at[pl.ds(s * RPT, RPT)])


def _dis_of(dp_ref):
    deg = dp_ref[0, :, 0:1] + dp_ref[1, :, 0:1] + 1.0
    return lax.rsqrt(deg)


def _tc1_body(x_ref, w1_ref, w2b_ref, root_ref, dp_ref, g1_ref, dis_ref,
              qg_ref, qacc):
    i = pl.program_id(0)
    dis = _dis_of(dp_ref)
    dis_ref[...] = dis
    xb = x_ref[...]
    h1 = jnp.dot(xb, w1_ref[...], preferred_element_type=jnp.float32)
    g1_ref[...] = h1 * dis
    ids = i * B + lax.broadcasted_iota(jnp.int32, (B, 1), 0)
    ohr = (ids == root_ref[...]).astype(jnp.float32)
    relu_x = jnp.maximum(xb, 0.0)
    contrib = lax.dot_general(ohr, relu_x, (((0,), (0,)), ((), ())),
                              preferred_element_type=jnp.float32)

    @pl.when(i == 0)
    def _():
        qacc[...] = contrib

    @pl.when(i > 0)
    def _():
        qacc[...] = qacc[...] + contrib

    qg_ref[...] = jnp.dot(qacc[...], w2b_ref[...],
                          preferred_element_type=jnp.float32)


def _tc2_body(p_ref, g1_ref, dis_in, b1_ref, qg_ref, batch_ref, w2a_ref,
              c1_ref, g2_ref):
    dis = dis_in[...]
    ssum = p_ref[0] + p_ref[1] + g1_ref[...]
    c1 = dis * ssum + b1_ref[...]
    c1_ref[...] = c1
    z = jnp.maximum(c1, 0.0)
    ohb = (batch_ref[...] == lax.broadcasted_iota(jnp.int32, (1, G), 1)
           ).astype(jnp.float32)
    h2 = (jnp.dot(z, w2a_ref[...], preferred_element_type=jnp.float32)
          + jnp.dot(ohb, qg_ref[...], preferred_element_type=jnp.float32))
    g2_ref[...] = h2 * dis


def _tc3_body(q_ref, g2_ref, dis_in, b2_ref, batch_ref, root_ref, c1_ref,
              out_ref, sums, cnt, r2acc):
    i = pl.program_id(0)
    dis = dis_in[...]
    c2 = dis * (q_ref[0] + q_ref[1] + g2_ref[...]) + b2_ref[...]
    r = jnp.maximum(c2, 0.0)
    ohb = (batch_ref[...] == lax.broadcasted_iota(jnp.int32, (1, G), 1)
           ).astype(jnp.float32)
    s_contrib = lax.dot_general(ohb, r, (((0,), (0,)), ((), ())),
                                preferred_element_type=jnp.float32)
    ones = jnp.ones((B, 1), jnp.float32)
    c_contrib = lax.dot_general(ohb, ones, (((0,), (0,)), ((), ())),
                                preferred_element_type=jnp.float32)
    ids = i * B + lax.broadcasted_iota(jnp.int32, (B, 1), 0)
    ohr = (ids == root_ref[...]).astype(jnp.float32)
    r2_contrib = lax.dot_general(ohr, c1_ref[...], (((0,), (0,)), ((), ())),
                                 preferred_element_type=jnp.float32)

    @pl.when(i == 0)
    def _():
        sums[...] = s_contrib
        cnt[...] = c_contrib
        r2acc[...] = r2_contrib

    @pl.when(i > 0)
    def _():
        sums[...] = sums[...] + s_contrib
        cnt[...] = cnt[...] + c_contrib
        r2acc[...] = r2acc[...] + r2_contrib

    counts = cnt[...]
    o1 = sums[...] / jnp.maximum(counts, 1.0)
    o2 = jnp.where(counts > 0.0, r2acc[...], 0.0)
    out_ref[...] = jnp.concatenate([o1, o2], axis=1)


def _tc1(x, W1, W2b, root2d, dp):
    return pl.pallas_call(
        _tc1_body,
        grid=(NB,),
        in_specs=[
            pl.BlockSpec((B, F), lambda i: (i, 0)),
            pl.BlockSpec((F, F), lambda i: (0, 0)),
            pl.BlockSpec((F, F), lambda i: (0, 0)),
            pl.BlockSpec((1, G), lambda i: (0, 0)),
            pl.BlockSpec((2, B, F), lambda i: (0, i, 0)),
        ],
        out_specs=[
            pl.BlockSpec((B, F), lambda i: (i, 0)),
            pl.BlockSpec((B, 1), lambda i: (i, 0)),
            pl.BlockSpec((G, F), lambda i: (0, 0)),
        ],
        out_shape=[
            jax.ShapeDtypeStruct((N, F), jnp.float32),
            jax.ShapeDtypeStruct((N, 1), jnp.float32),
            jax.ShapeDtypeStruct((G, F), jnp.float32),
        ],
        scratch_shapes=[pltpu.VMEM((G, F), jnp.float32)],
    )(x, W1, W2b, root2d, dp)


def _tc2(p, g1, disb, b1_2d, qg, batch2d, W2a):
    return pl.pallas_call(
        _tc2_body,
        grid=(NB,),
        in_specs=[
            pl.BlockSpec((2, B, F), lambda i: (0, i, 0)),
            pl.BlockSpec((B, F), lambda i: (i, 0)),
            pl.BlockSpec((B, 1), lambda i: (i, 0)),
            pl.BlockSpec((1, F), lambda i: (0, 0)),
            pl.BlockSpec((G, F), lambda i: (0, 0)),
            pl.BlockSpec((B, 1), lambda i: (i, 0)),
            pl.BlockSpec((F, F), lambda i: (0, 0)),
        ],
        out_specs=[
            pl.BlockSpec((B, F), lambda i: (i, 0)),
            pl.BlockSpec((B, F), lambda i: (i, 0)),
        ],
        out_shape=[
            jax.ShapeDtypeStruct((N, F), jnp.float32),
            jax.ShapeDtypeStruct((N, F), jnp.float32),
        ],
    )(p, g1, disb, b1_2d, qg, batch2d, W2a)


def _tc3(q, g2, disb, b2_2d, batch2d, root2d, c1):
    return pl.pallas_call(
        _tc3_body,
        grid=(NB,),
        in_specs=[
            pl.BlockSpec((2, B, F), lambda i: (0, i, 0)),
            pl.BlockSpec((B, F), lambda i: (i, 0)),
            pl.BlockSpec((B, 1), lambda i: (i, 0)),
            pl.BlockSpec((1, F), lambda i: (0, 0)),
            pl.BlockSpec((B, 1), lambda i: (i, 0)),
            pl.BlockSpec((1, G), lambda i: (0, 0)),
            pl.BlockSpec((B, F), lambda i: (i, 0)),
        ],
        out_specs=pl.BlockSpec((G, 2 * F), lambda i: (0, 0)),
        out_shape=jax.ShapeDtypeStruct((G, 2 * F), jnp.float32),
        scratch_shapes=[
            pltpu.VMEM((G, F), jnp.float32),
            pltpu.VMEM((G, 1), jnp.float32),
            pltpu.VMEM((G, F), jnp.float32),
        ],
    )(q, g2, disb, b2_2d, batch2d, root2d, c1)


@jax.jit
def kernel(x, edge_index, batch, root_index, W1, b1, W2, b2):
    row = edge_index[0].astype(jnp.int32)
    col = edge_index[1].astype(jnp.int32)
    pad = EPAD - E
    lane = jnp.arange(CK, dtype=jnp.int32)
    padrow = jnp.tile(lane * 77, pad // CK)
    padcol = jnp.tile(N + lane, pad // CK)
    rowp = jnp.concatenate([row, padrow])
    colp = jnp.concatenate([col, padcol])
    rowp = rowp.reshape(NCHUNKS, CK)
    colp = colp.reshape(NCHUNKS, CK)
    colp_deg = colp.reshape(TILES, CPT, CK)

    vones = jnp.zeros((CK, F), jnp.float32).at[:, 0].set(1.0)
    zerosF = jnp.zeros((RPT, F), jnp.float32)

    root2d = root_index.astype(jnp.int32).reshape(1, G)
    batch2d = batch.astype(jnp.int32).reshape(N, 1)
    b1_2d = b1.reshape(1, F)
    b2_2d = b2.reshape(1, F)
    W2a = W2[:F]
    W2b = W2[F:]

    dp = _deg_kernel(colp_deg, vones, zerosF)
    g1, disb, qg = _tc1(x, W1, W2b, root2d, dp)
    p1 = _agg_kernel(g1, rowp, colp, zerosF)
    c1, g2 = _tc2(p1, g1, disb, b1_2d, qg, batch2d, W2a)
    p2 = _agg_kernel(g2, rowp, colp, zerosF)
    out = _tc3(p2, g2, disb, b2_2d, batch2d, root2d, c1)
    return out


# speedup vs baseline: 1.2653x; 1.0033x over previous
import functools

import jax
import jax.numpy as jnp
from jax import lax
from jax.experimental import pallas as pl
from jax.experimental.pallas import tpu as pltpu
from jax.experimental.pallas import tpu_sc as plsc

N = 10000
E = 320000
F = 128
G = 128

NPAD = 10240
TILES = 32
CK = 128
CPT = 80
EPAD = TILES * CPT * CK
RPT = NPAD // 16

B = 2000
NB = N // B

_mesh = plsc.VectorSubcoreMesh(core_axis_name="c", subcore_axis_name="s")


@functools.partial(
    pl.kernel,
    out_type=jax.ShapeDtypeStruct((2, NPAD, F), jnp.float32),
    mesh=_mesh,
    scratch_types=[
        pltpu.VMEM((CPT, CK), jnp.int32),
        pltpu.VMEM((CK, F), jnp.float32),
        pltpu.MemorySpace.VMEM_SHARED((NPAD, F), jnp.float32),
    ],
)
def _deg_kernel(colp_hbm, vones_hbm, zeros_hbm, out_hbm, colv, vones, dacc):
    c = lax.axis_index("c")
    s = lax.axis_index("s")
    wid = c * 16 + s
    pltpu.sync_copy(zeros_hbm, dacc.at[pl.ds(s * RPT, RPT)])
    pltpu.sync_copy(colp_hbm.at[wid], colv)
    pltpu.sync_copy(vones_hbm, vones)
    plsc.subcore_barrier()

    def body(j, carry):
        pltpu.sync_copy(vones, dacc.at[colv.at[j]], add=True)
        return carry

    lax.fori_loop(0, CPT, body, 0)
    plsc.subcore_barrier()
    pltpu.sync_copy(dacc.at[pl.ds(s * RPT, RPT)],
                    out_hbm.at[c].at[pl.ds(s * RPT, RPT)])


QTR = 40
CPT0 = 80
CPT1 = 80
NCHUNKS = 16 * (CPT0 + CPT1)


@functools.partial(
    pl.kernel,
    out_type=jax.ShapeDtypeStruct((2, NPAD, F), jnp.float32),
    mesh=_mesh,
    scratch_types=[
        pltpu.VMEM((QTR, CK), jnp.int32),
        pltpu.VMEM((QTR, CK), jnp.int32),
        pltpu.VMEM((CK, F), jnp.float32),
        pltpu.VMEM((CK, F), jnp.float32),
        pltpu.MemorySpace.VMEM_SHARED((NPAD, F), jnp.float32),
        pltpu.SemaphoreType.DMA,
        pltpu.SemaphoreType.DMA,
    ],
)
def _agg_kernel(g_hbm, rowp_hbm, colp_hbm, zeros_hbm, out_hbm,
                rowv, colv, bufa, bufb, acc, sema, semb):
    c = lax.axis_index("c")
    s = lax.axis_index("s")
    base = jnp.where(c == 0, s * CPT0, 16 * CPT0 + s * CPT1)
    nph = jnp.where(c == 0, CPT0 // QTR, CPT1 // QTR)
    pltpu.sync_copy(zeros_hbm, acc.at[pl.ds(s * RPT, RPT)])
    plsc.subcore_barrier()

    def wait_g(sem):
        pltpu.make_async_copy(g_hbm.at[pl.ds(0, CK)], bufa, sem).wait()

    for q in range(max(CPT0, CPT1) // QTR):
        @pl.when(q < nph)
        def _():
            off = base + q * QTR
            pltpu.sync_copy(rowp_hbm.at[pl.ds(off, QTR)], rowv)
            pltpu.sync_copy(colp_hbm.at[pl.ds(off, QTR)], colv)

            pltpu.async_copy(g_hbm.at[rowv.at[0]], bufa, sema)

            def body(i, carry):
                ja = 2 * i
                jb = 2 * i + 1
                pltpu.async_copy(g_hbm.at[rowv.at[jb]], bufb, semb)
                wait_g(sema)
                pltpu.sync_copy(bufa, acc.at[colv.at[ja]], add=True)

                @pl.when(i < QTR // 2 - 1)
                def _():
                    pltpu.async_copy(g_hbm.at[rowv.at[ja + 2]], bufa, sema)

                wait_g(semb)
                pltpu.sync_copy(bufb, acc.at[colv.at[jb]], add=True)
                return carry

            lax.fori_loop(0, QTR // 2, body, 0)

    plsc.subcore_barrier()
    pltpu.sync_copy(acc.at[pl.ds(s * RPT, RPT)],
                    out_hbm.at[c].at[pl.ds(s * RPT, RPT)])


def _dis_of(dp_ref):
    deg = dp_ref[0, :, 0:1] + dp_ref[1, :, 0:1] + 1.0
    return lax.rsqrt(deg)


def _tc1_body(x_ref, w1_ref, w2b_ref, root_ref, dp_ref, g1_ref, dis_ref,
              qg_ref, qacc):
    i = pl.program_id(0)
    dis = _dis_of(dp_ref)
    dis_ref[...] = dis
    xb = x_ref[...]
    h1 = jnp.dot(xb, w1_ref[...], preferred_element_type=jnp.float32)
    g1_ref[...] = h1 * dis
    ids = i * B + lax.broadcasted_iota(jnp.int32, (B, 1), 0)
    ohr = (ids == root_ref[...]).astype(jnp.float32)
    relu_x = jnp.maximum(xb, 0.0)
    contrib = lax.dot_general(ohr, relu_x, (((0,), (0,)), ((), ())),
                              preferred_element_type=jnp.float32)

    @pl.when(i == 0)
    def _():
        qacc[...] = contrib

    @pl.when(i > 0)
    def _():
        qacc[...] = qacc[...] + contrib

    qg_ref[...] = jnp.dot(qacc[...], w2b_ref[...],
                          preferred_element_type=jnp.float32)


def _tc2_body(p_ref, g1_ref, dis_in, b1_ref, qg_ref, batch_ref, w2a_ref,
              c1_ref, g2_ref):
    dis = dis_in[...]
    ssum = p_ref[0] + p_ref[1] + g1_ref[...]
    c1 = dis * ssum + b1_ref[...]
    c1_ref[...] = c1
    z = jnp.maximum(c1, 0.0)
    ohb = (batch_ref[...] == lax.broadcasted_iota(jnp.int32, (1, G), 1)
           ).astype(jnp.float32)
    h2 = (jnp.dot(z, w2a_ref[...], preferred_element_type=jnp.float32)
          + jnp.dot(ohb, qg_ref[...], preferred_element_type=jnp.float32))
    g2_ref[...] = h2 * dis


def _tc3_body(q_ref, g2_ref, dis_in, b2_ref, batch_ref, root_ref, c1_ref,
              out_ref, sums, cnt, r2acc):
    i = pl.program_id(0)
    dis = dis_in[...]
    c2 = dis * (q_ref[0] + q_ref[1] + g2_ref[...]) + b2_ref[...]
    r = jnp.maximum(c2, 0.0)
    ohb = (batch_ref[...] == lax.broadcasted_iota(jnp.int32, (1, G), 1)
           ).astype(jnp.float32)
    s_contrib = lax.dot_general(ohb, r, (((0,), (0,)), ((), ())),
                                preferred_element_type=jnp.float32)
    ones = jnp.ones((B, 1), jnp.float32)
    c_contrib = lax.dot_general(ohb, ones, (((0,), (0,)), ((), ())),
                                preferred_element_type=jnp.float32)
    ids = i * B + lax.broadcasted_iota(jnp.int32, (B, 1), 0)
    ohr = (ids == root_ref[...]).astype(jnp.float32)
    r2_contrib = lax.dot_general(ohr, c1_ref[...], (((0,), (0,)), ((), ())),
                                 preferred_element_type=jnp.float32)

    @pl.when(i == 0)
    def _():
        sums[...] = s_contrib
        cnt[...] = c_contrib
        r2acc[...] = r2_contrib

    @pl.when(i > 0)
    def _():
        sums[...] = sums[...] + s_contrib
        cnt[...] = cnt[...] + c_contrib
        r2acc[...] = r2acc[...] + r2_contrib

    counts = cnt[...]
    o1 = sums[...] / jnp.maximum(counts, 1.0)
    o2 = jnp.where(counts > 0.0, r2acc[...], 0.0)
    out_ref[...] = jnp.concatenate([o1, o2], axis=1)


def _tc1(x, W1, W2b, root2d, dp):
    return pl.pallas_call(
        _tc1_body,
        grid=(NB,),
        in_specs=[
            pl.BlockSpec((B, F), lambda i: (i, 0)),
            pl.BlockSpec((F, F), lambda i: (0, 0)),
            pl.BlockSpec((F, F), lambda i: (0, 0)),
            pl.BlockSpec((1, G), lambda i: (0, 0)),
            pl.BlockSpec((2, B, F), lambda i: (0, i, 0)),
        ],
        out_specs=[
            pl.BlockSpec((B, F), lambda i: (i, 0)),
            pl.BlockSpec((B, 1), lambda i: (i, 0)),
            pl.BlockSpec((G, F), lambda i: (0, 0)),
        ],
        out_shape=[
            jax.ShapeDtypeStruct((N, F), jnp.float32),
            jax.ShapeDtypeStruct((N, 1), jnp.float32),
            jax.ShapeDtypeStruct((G, F), jnp.float32),
        ],
        scratch_shapes=[pltpu.VMEM((G, F), jnp.float32)],
    )(x, W1, W2b, root2d, dp)


def _tc2(p, g1, disb, b1_2d, qg, batch2d, W2a):
    return pl.pallas_call(
        _tc2_body,
        grid=(NB,),
        in_specs=[
            pl.BlockSpec((2, B, F), lambda i: (0, i, 0)),
            pl.BlockSpec((B, F), lambda i: (i, 0)),
            pl.BlockSpec((B, 1), lambda i: (i, 0)),
            pl.BlockSpec((1, F), lambda i: (0, 0)),
            pl.BlockSpec((G, F), lambda i: (0, 0)),
            pl.BlockSpec((B, 1), lambda i: (i, 0)),
            pl.BlockSpec((F, F), lambda i: (0, 0)),
        ],
        out_specs=[
            pl.BlockSpec((B, F), lambda i: (i, 0)),
            pl.BlockSpec((B, F), lambda i: (i, 0)),
        ],
        out_shape=[
            jax.ShapeDtypeStruct((N, F), jnp.float32),
            jax.ShapeDtypeStruct((N, F), jnp.float32),
        ],
    )(p, g1, disb, b1_2d, qg, batch2d, W2a)


def _tc3(q, g2, disb, b2_2d, batch2d, root2d, c1):
    return pl.pallas_call(
        _tc3_body,
        grid=(NB,),
        in_specs=[
            pl.BlockSpec((2, B, F), lambda i: (0, i, 0)),
            pl.BlockSpec((B, F), lambda i: (i, 0)),
            pl.BlockSpec((B, 1), lambda i: (i, 0)),
            pl.BlockSpec((1, F), lambda i: (0, 0)),
            pl.BlockSpec((B, 1), lambda i: (i, 0)),
            pl.BlockSpec((1, G), lambda i: (0, 0)),
            pl.BlockSpec((B, F), lambda i: (i, 0)),
        ],
        out_specs=pl.BlockSpec((G, 2 * F), lambda i: (0, 0)),
        out_shape=jax.ShapeDtypeStruct((G, 2 * F), jnp.float32),
        scratch_shapes=[
            pltpu.VMEM((G, F), jnp.float32),
            pltpu.VMEM((G, 1), jnp.float32),
            pltpu.VMEM((G, F), jnp.float32),
        ],
    )(q, g2, disb, b2_2d, batch2d, root2d, c1)


@jax.jit
def kernel(x, edge_index, batch, root_index, W1, b1, W2, b2):
    row = edge_index[0].astype(jnp.int32)
    col = edge_index[1].astype(jnp.int32)
    pad = EPAD - E
    lane = jnp.arange(CK, dtype=jnp.int32)
    padrow = jnp.tile(lane * 77, pad // CK)
    padcol = jnp.tile(N + lane, pad // CK)
    rowp = jnp.concatenate([row, padrow])
    colp = jnp.concatenate([col, padcol])
    rowp = rowp.reshape(NCHUNKS, CK)
    colp = colp.reshape(NCHUNKS, CK)
    colp_deg = colp.reshape(TILES, CPT, CK)

    vones = jnp.zeros((CK, F), jnp.float32).at[:, 0].set(1.0)
    zerosF = jnp.zeros((RPT, F), jnp.float32)

    root2d = root_index.astype(jnp.int32).reshape(1, G)
    batch2d = batch.astype(jnp.int32).reshape(N, 1)
    b1_2d = b1.reshape(1, F)
    b2_2d = b2.reshape(1, F)
    W2a = W2[:F]
    W2b = W2[F:]

    dp = _deg_kernel(colp_deg, vones, zerosF)
    g1, disb, qg = _tc1(x, W1, W2b, root2d, dp)
    p1 = _agg_kernel(g1, rowp, colp, zerosF)
    c1, g2 = _tc2(p1, g1, disb, b1_2d, qg, batch2d, W2a)
    p2 = _agg_kernel(g2, rowp, colp, zerosF)
    out = _tc3(p2, g2, disb, b2_2d, batch2d, root2d, c1)
    return out
